# baseline (speedup 1.0000x reference)
import jax
import jax.numpy as jnp
from jax import lax
from jax.experimental import pallas as pl
from jax.experimental.pallas import tpu as pltpu
from jax.experimental.pallas import tpu_sc as plsc

N_USERS = 25000
N_ITEMS = 25000
N_NODES = N_USERS + N_ITEMS
D = 64
DQ = 16
NQ = D // DQ
NNZ = 800000

NS = 16
NC = 2
NROW = 50048
STRIPE = NROW // NS
E_GRP = 128
GRPS = 4
E_CHUNK = GRPS * E_GRP
CHUNKS = 100
EDGES_PER_TILE = CHUNKS * E_CHUNK
NNZ_PAD = EDGES_PER_TILE * NS
CROWS = NNZ_PAD // E_GRP
TROWS = CHUNKS * GRPS


_PRE_R = 2000


def _pre_body(t_ref, emb_ref, wa_ref, ba_ref, wp_ref, o0, o1, o2, o3):
    ph = t_ref[:] * wa_ref[:] + ba_ref[:]
    lane = lax.broadcasted_iota(jnp.int32, ph.shape, 1)
    val = jnp.where(lane == 0, ph, jnp.sin(ph))
    res = lax.dot_general(val, wp_ref[:], (((1,), (1,)), ((), ())),
                          preferred_element_type=jnp.float32)
    res = res + emb_ref[:]
    o0[:] = res[:, 0 * DQ:1 * DQ]
    o1[:] = res[:, 1 * DQ:2 * DQ]
    o2[:] = res[:, 2 * DQ:3 * DQ]
    o3[:] = res[:, 3 * DQ:4 * DQ]


def _preamble(t_all, emb_all, wa, ba, wproj):
    nb = N_NODES // _PRE_R
    qshape = jax.ShapeDtypeStruct((NROW, DQ), jnp.float32)
    qspec = pl.BlockSpec((_PRE_R, DQ), lambda i: (i, 0))
    return pl.pallas_call(
        _pre_body,
        grid=(nb,),
        in_specs=[
            pl.BlockSpec((_PRE_R, 1), lambda i: (i, 0)),
            pl.BlockSpec((_PRE_R, D), lambda i: (i, 0)),
            pl.BlockSpec((1, D), lambda i: (0, 0)),
            pl.BlockSpec((1, D), lambda i: (0, 0)),
            pl.BlockSpec((D, D), lambda i: (0, 0)),
        ],
        out_specs=[qspec] * NQ,
        out_shape=[qshape] * NQ,
    )(t_all, emb_all, wa, ba, wproj)


def _spmm_body(x0, cv4, rows2d, zstripe,
               o1, o2, o3,
               rowsv, ca, cb, ga, gb, acc, sem_i, sem_g, sem_s):
    c = lax.axis_index("c")
    s = lax.axis_index("s")
    row0 = s * STRIPE
    erow = s * TROWS

    pltpu.sync_copy(rows2d.at[pl.ds(erow, TROWS)], rowsv)

    def run_pass(src, dst, p):
        qrow = ((c * 2 + p) * NS + s) * (CHUNKS * 2 * GRPS)

        def idx_fetch(k, cbuf):
            pltpu.async_copy(cv4.at[pl.ds(qrow + k * 2 * GRPS, 2 * GRPS)],
                             cbuf, sem_i)

        def idx_wait(k, cbuf):
            pltpu.make_async_copy(cv4.at[pl.ds(qrow + k * 2 * GRPS,
                                               2 * GRPS)],
                                  cbuf, sem_i).wait()

        def g_issue(k, cbuf, gbuf):
            for j in range(GRPS):
                pltpu.async_copy(src.at[cbuf.at[j]], gbuf.at[j], sem_g)

        def g_wait(k, cbuf, gbuf):
            for j in range(GRPS):
                pltpu.make_async_copy(src.at[cbuf.at[j]], gbuf.at[j],
                                      sem_g).wait()

        def scale(k, cbuf, gbuf):
            for j in range(GRPS):
                def qb(q, _, j=j):
                    vv = plsc.bitcast(cbuf[GRPS + j, pl.ds(q * 16, 16)],
                                      jnp.float32)
                    for i in range(16):
                        e = q * 16 + i
                        gbuf[j, e, :] = gbuf[j, e, :] * vv[i]
                    return 0
                lax.fori_loop(0, E_GRP // 16, qb, 0)

        def s_issue(k, gbuf):
            for j in range(GRPS):
                pltpu.async_copy(gbuf.at[j], acc.at[rowsv.at[k * GRPS + j]],
                                 sem_s, add=True)

        def s_wait(k, gbuf):
            for j in range(GRPS):
                pltpu.make_async_copy(gbuf.at[j],
                                      acc.at[rowsv.at[k * GRPS + j]],
                                      sem_s).wait()

        def body(k, cur, nxt, first=False, last=False):
            cc, cg = cur
            nc, ng = nxt
            g_wait(k, cc, cg)
            if not last:
                idx_fetch(k + 1, nc)
            scale(k, cc, cg)
            if not first:
                s_wait(k - 1, ng)
            if not last:
                idx_wait(k + 1, nc)
                g_issue(k + 1, nc, ng)
            s_issue(k, cg)

        A = (ca, ga)
        B = (cb, gb)
        idx_fetch(0, ca)
        idx_wait(0, ca)
        g_issue(0, ca, ga)
        body(0, A, B, first=True)
        def steady(k2, carry):
            k = 2 * k2 + 1
            body(k, B, A)
            body(k + 1, A, B)
            return carry
        lax.fori_loop(0, (CHUNKS - 2) // 2, steady, 0)
        body(CHUNKS - 1, B, A, last=True)
        s_wait(CHUNKS - 1, gb)

    for src, dst in ((x0, o1), (o1, o2), (o2, o3)):
        def pbody(p, carry, src=src, dst=dst):
            pltpu.sync_copy(zstripe, acc.at[pl.ds(row0, STRIPE)])
            plsc.subcore_barrier()
            run_pass(src, dst, p)
            plsc.subcore_barrier()
            pltpu.sync_copy(
                acc.at[pl.ds(row0, STRIPE)],
                dst.at[pl.ds((c * 2 + p) * NROW + row0, STRIPE)])
            return carry
        lax.fori_loop(0, 2, pbody, 0)


def _spmm3(x0, cv4, rows2d, zstripe):
    mesh = plsc.VectorSubcoreMesh(core_axis_name="c", subcore_axis_name="s")
    xshape = jax.ShapeDtypeStruct((NQ * NROW, DQ), jnp.float32)
    f = pl.kernel(
        _spmm_body,
        out_type=(xshape, xshape, xshape),
        mesh=mesh,
        scratch_types=[
            pltpu.VMEM((TROWS, E_GRP), jnp.int32),
            pltpu.VMEM((2 * GRPS, E_GRP), jnp.int32),
            pltpu.VMEM((2 * GRPS, E_GRP), jnp.int32),
            pltpu.VMEM((GRPS, E_GRP, DQ), jnp.float32),
            pltpu.VMEM((GRPS, E_GRP, DQ), jnp.float32),
            pltpu.VMEM_SHARED((NROW, DQ), jnp.float32),
            pltpu.SemaphoreType.DMA,
            pltpu.SemaphoreType.DMA,
            pltpu.SemaphoreType.DMA,
        ],
        compiler_params=pltpu.CompilerParams(use_tc_tiling_on_sc=False,
                                             needs_layout_passes=False),
    )
    return f(x0, cv4, rows2d, zstripe)


_M_R = 1000


def _mean_body(a, b, c, d, ou, oi):
    i = pl.program_id(0)
    s = [(a[q] + b[q] + c[q] + d[q]) * 0.25 for q in range(NQ)]
    res = jnp.concatenate(s, axis=1)

    @pl.when(i < (N_USERS // _M_R))
    def _():
        ou[:] = res

    @pl.when(i >= (N_USERS // _M_R))
    def _():
        oi[:] = res


def _mean4(a, b, c, d):
    nu = N_USERS // _M_R
    ispec = pl.BlockSpec((NQ, _M_R, DQ), lambda i: (0, i, 0))
    uspec = pl.BlockSpec((_M_R, D), lambda i: (jnp.minimum(i, nu - 1), 0))
    ospec = pl.BlockSpec((_M_R, D), lambda i: (jnp.maximum(i - nu, 0), 0))
    oshape = jax.ShapeDtypeStruct((N_USERS, D), jnp.float32)
    return pl.pallas_call(
        _mean_body,
        grid=(N_NODES // _M_R,),
        in_specs=[ispec] * 4,
        out_specs=[uspec, ospec],
        out_shape=[oshape, oshape],
    )(a, b, c, d)


def kernel(user_recency, item_recency, adj_vals, user_emb, item_emb,
           w0, b0, w, b, Wproj, edge_index):
    t_all = jnp.concatenate([user_recency, item_recency]).reshape(N_NODES, 1)
    emb_all = jnp.concatenate([user_emb, item_emb], axis=0)
    wa = jnp.concatenate([w0, w]).reshape(1, D)
    ba = jnp.concatenate([b0, b]).reshape(1, D)

    xq = _preamble(t_all, emb_all, wa, ba, Wproj)
    x0 = jnp.concatenate(xq, axis=0)

    rows = edge_index[0].astype(jnp.int32)
    cols = edge_index[1].astype(jnp.int32)
    perm = jnp.argsort(rows // (NROW // NQ))
    rows = rows[perm]
    cols = cols[perm]
    adj_vals = adj_vals[perm]
    pad = NNZ_PAD - NNZ
    ipad = jnp.zeros((pad,), jnp.int32)
    rows2d = jnp.concatenate([rows, ipad]).reshape(CROWS, E_GRP)
    colsp = jnp.concatenate([cols, ipad])
    valsp = jnp.concatenate([adj_vals, jnp.zeros((pad,), jnp.float32)])
    cshift = (colsp[None, :]
              + (jnp.arange(NQ, dtype=jnp.int32) * NROW)[:, None])
    cshift = cshift.reshape(NQ, NS, CHUNKS, GRPS, E_GRP)
    vi = lax.bitcast_convert_type(valsp, jnp.int32)
    vi = jnp.broadcast_to(vi.reshape(1, NS, CHUNKS, GRPS, E_GRP),
                          (NQ, NS, CHUNKS, GRPS, E_GRP))
    cv4 = jnp.concatenate([cshift, vi], axis=3)
    cv4 = cv4.reshape(NQ * NS * CHUNKS * 2 * GRPS, E_GRP)
    zstripe = jnp.zeros((STRIPE, DQ), jnp.float32)

    x1, x2, x3 = _spmm3(x0, cv4, rows2d, zstripe)

    q3 = lambda v: v.reshape(NQ, NROW, DQ)
    u_out, i_out = _mean4(q3(x0), q3(x1), q3(x2), q3(x3))
    return u_out, i_out


# speedup vs baseline: 1.4533x; 1.4533x over previous
import jax
import jax.numpy as jnp
from jax import lax
from jax.experimental import pallas as pl
from jax.experimental.pallas import tpu as pltpu
from jax.experimental.pallas import tpu_sc as plsc

N_USERS = 25000
N_ITEMS = 25000
N_NODES = N_USERS + N_ITEMS
D = 64
DQ = 16
NQ = D // DQ
NNZ = 800000

NS = 16
NC = 2
NROW = 50048
STRIPE = NROW // NS
E_GRP = 128
GRPS = 4
E_CHUNK = GRPS * E_GRP
CHUNKS = 100
EDGES_PER_TILE = CHUNKS * E_CHUNK
NNZ_PAD = EDGES_PER_TILE * NS
CROWS = NNZ_PAD // E_GRP
TROWS = CHUNKS * GRPS


_PRE_R = 2000


def _pre_body(t_ref, emb_ref, wa_ref, ba_ref, wp_ref, o0, o1, o2, o3):
    ph = t_ref[:] * wa_ref[:] + ba_ref[:]
    lane = lax.broadcasted_iota(jnp.int32, ph.shape, 1)
    val = jnp.where(lane == 0, ph, jnp.sin(ph))
    res = lax.dot_general(val, wp_ref[:], (((1,), (1,)), ((), ())),
                          preferred_element_type=jnp.float32)
    res = res + emb_ref[:]
    o0[:] = res[:, 0 * DQ:1 * DQ]
    o1[:] = res[:, 1 * DQ:2 * DQ]
    o2[:] = res[:, 2 * DQ:3 * DQ]
    o3[:] = res[:, 3 * DQ:4 * DQ]


def _preamble(t_all, emb_all, wa, ba, wproj):
    nb = N_NODES // _PRE_R
    qshape = jax.ShapeDtypeStruct((NROW, DQ), jnp.float32)
    qspec = pl.BlockSpec((_PRE_R, DQ), lambda i: (i, 0))
    return pl.pallas_call(
        _pre_body,
        grid=(nb,),
        in_specs=[
            pl.BlockSpec((_PRE_R, 1), lambda i: (i, 0)),
            pl.BlockSpec((_PRE_R, D), lambda i: (i, 0)),
            pl.BlockSpec((1, D), lambda i: (0, 0)),
            pl.BlockSpec((1, D), lambda i: (0, 0)),
            pl.BlockSpec((D, D), lambda i: (0, 0)),
        ],
        out_specs=[qspec] * NQ,
        out_shape=[qshape] * NQ,
    )(t_all, emb_all, wa, ba, wproj)


def _spmm_body(x0, cv4, rows2d, zstripe,
               o1, o2, o3,
               rowsv, ca, cb, ga, gb, acc, sem_i, sem_g, sem_s):
    c = lax.axis_index("c")
    s = lax.axis_index("s")
    row0 = s * STRIPE
    erow = s * TROWS

    pltpu.sync_copy(rows2d.at[pl.ds(erow, TROWS)], rowsv)

    def run_pass(src, dst, p):
        qoff = (c * 2 + p) * NROW
        qrow = s * (CHUNKS * 2 * GRPS)

        def idx_fetch(k, cbuf):
            pltpu.async_copy(cv4.at[pl.ds(qrow + k * 2 * GRPS, 2 * GRPS)],
                             cbuf, sem_i)

        def idx_wait(k, cbuf):
            pltpu.make_async_copy(cv4.at[pl.ds(qrow + k * 2 * GRPS,
                                               2 * GRPS)],
                                  cbuf, sem_i).wait()

        def shift(cbuf):
            for j in range(GRPS):
                for qq in range(E_GRP // 16):
                    cbuf[j, pl.ds(qq * 16, 16)] = (
                        cbuf[j, pl.ds(qq * 16, 16)] + qoff)

        def g_issue(k, cbuf, gbuf):
            for j in range(GRPS):
                pltpu.async_copy(src.at[cbuf.at[j]], gbuf.at[j], sem_g)

        def g_wait(k, cbuf, gbuf):
            for j in range(GRPS):
                pltpu.make_async_copy(src.at[cbuf.at[j]], gbuf.at[j],
                                      sem_g).wait()

        def scale(k, cbuf, gbuf):
            for j in range(GRPS):
                def qb(q, _, j=j):
                    vv = plsc.bitcast(cbuf[GRPS + j, pl.ds(q * 16, 16)],
                                      jnp.float32)
                    for i in range(16):
                        e = q * 16 + i
                        gbuf[j, e, :] = gbuf[j, e, :] * vv[i]
                    return 0
                lax.fori_loop(0, E_GRP // 16, qb, 0)

        def s_issue(k, gbuf):
            for j in range(GRPS):
                pltpu.async_copy(gbuf.at[j], acc.at[rowsv.at[k * GRPS + j]],
                                 sem_s, add=True)

        def s_wait(k, gbuf):
            for j in range(GRPS):
                pltpu.make_async_copy(gbuf.at[j],
                                      acc.at[rowsv.at[k * GRPS + j]],
                                      sem_s).wait()

        def body(k, cur, nxt, first=False, last=False):
            cc, cg = cur
            nc, ng = nxt
            g_wait(k, cc, cg)
            if not last:
                idx_fetch(k + 1, nc)
            scale(k, cc, cg)
            if not first:
                s_wait(k - 1, ng)
            if not last:
                idx_wait(k + 1, nc)
                shift(nc)
                g_issue(k + 1, nc, ng)
            s_issue(k, cg)

        A = (ca, ga)
        B = (cb, gb)
        idx_fetch(0, ca)
        idx_wait(0, ca)
        shift(ca)
        g_issue(0, ca, ga)
        body(0, A, B, first=True)
        def steady(k2, carry):
            k = 2 * k2 + 1
            body(k, B, A)
            body(k + 1, A, B)
            return carry
        lax.fori_loop(0, (CHUNKS - 2) // 2, steady, 0)
        body(CHUNKS - 1, B, A, last=True)
        s_wait(CHUNKS - 1, gb)

    for src, dst in ((x0, o1), (o1, o2), (o2, o3)):
        def pbody(p, carry, src=src, dst=dst):
            pltpu.sync_copy(zstripe, acc.at[pl.ds(row0, STRIPE)])
            plsc.subcore_barrier()
            run_pass(src, dst, p)
            plsc.subcore_barrier()
            pltpu.sync_copy(
                acc.at[pl.ds(row0, STRIPE)],
                dst.at[pl.ds((c * 2 + p) * NROW + row0, STRIPE)])
            return carry
        lax.fori_loop(0, 2, pbody, 0)


def _spmm3(x0, cv4, rows2d, zstripe):
    mesh = plsc.VectorSubcoreMesh(core_axis_name="c", subcore_axis_name="s")
    xshape = jax.ShapeDtypeStruct((NQ * NROW, DQ), jnp.float32)
    f = pl.kernel(
        _spmm_body,
        out_type=(xshape, xshape, xshape),
        mesh=mesh,
        scratch_types=[
            pltpu.VMEM((TROWS, E_GRP), jnp.int32),
            pltpu.VMEM((2 * GRPS, E_GRP), jnp.int32),
            pltpu.VMEM((2 * GRPS, E_GRP), jnp.int32),
            pltpu.VMEM((GRPS, E_GRP, DQ), jnp.float32),
            pltpu.VMEM((GRPS, E_GRP, DQ), jnp.float32),
            pltpu.VMEM_SHARED((NROW, DQ), jnp.float32),
            pltpu.SemaphoreType.DMA,
            pltpu.SemaphoreType.DMA,
            pltpu.SemaphoreType.DMA,
        ],
        compiler_params=pltpu.CompilerParams(use_tc_tiling_on_sc=False,
                                             needs_layout_passes=False),
    )
    return f(x0, cv4, rows2d, zstripe)


_M_R = 1000


def _mean_body(a, b, c, d, ou, oi):
    i = pl.program_id(0)
    s = [(a[q] + b[q] + c[q] + d[q]) * 0.25 for q in range(NQ)]
    res = jnp.concatenate(s, axis=1)

    @pl.when(i < (N_USERS // _M_R))
    def _():
        ou[:] = res

    @pl.when(i >= (N_USERS // _M_R))
    def _():
        oi[:] = res


def _mean4(a, b, c, d):
    nu = N_USERS // _M_R
    ispec = pl.BlockSpec((NQ, _M_R, DQ), lambda i: (0, i, 0))
    uspec = pl.BlockSpec((_M_R, D), lambda i: (jnp.minimum(i, nu - 1), 0))
    ospec = pl.BlockSpec((_M_R, D), lambda i: (jnp.maximum(i - nu, 0), 0))
    oshape = jax.ShapeDtypeStruct((N_USERS, D), jnp.float32)
    return pl.pallas_call(
        _mean_body,
        grid=(N_NODES // _M_R,),
        in_specs=[ispec] * 4,
        out_specs=[uspec, ospec],
        out_shape=[oshape, oshape],
    )(a, b, c, d)


def kernel(user_recency, item_recency, adj_vals, user_emb, item_emb,
           w0, b0, w, b, Wproj, edge_index):
    t_all = jnp.concatenate([user_recency, item_recency]).reshape(N_NODES, 1)
    emb_all = jnp.concatenate([user_emb, item_emb], axis=0)
    wa = jnp.concatenate([w0, w]).reshape(1, D)
    ba = jnp.concatenate([b0, b]).reshape(1, D)

    xq = _preamble(t_all, emb_all, wa, ba, Wproj)
    x0 = jnp.concatenate(xq, axis=0)

    rows = edge_index[0].astype(jnp.int32)
    cols = edge_index[1].astype(jnp.int32)
    pad = NNZ_PAD - NNZ
    ipad = jnp.zeros((pad,), jnp.int32)
    rows2d = jnp.concatenate([rows, ipad]).reshape(CROWS, E_GRP)
    colsp = jnp.concatenate([cols, ipad])
    valsp = jnp.concatenate([adj_vals, jnp.zeros((pad,), jnp.float32)])
    cs = colsp.reshape(NS, CHUNKS, GRPS, E_GRP)
    vi = lax.bitcast_convert_type(valsp, jnp.int32)
    vi = vi.reshape(NS, CHUNKS, GRPS, E_GRP)
    cv4 = jnp.concatenate([cs, vi], axis=2)
    cv4 = cv4.reshape(NS * CHUNKS * 2 * GRPS, E_GRP)
    zstripe = jnp.zeros((STRIPE, DQ), jnp.float32)

    x1, x2, x3 = _spmm3(x0, cv4, rows2d, zstripe)

    q3 = lambda v: v.reshape(NQ, NROW, DQ)
    u_out, i_out = _mean4(q3(x0), q3(x1), q3(x2), q3(x3))
    return u_out, i_out


# speedup vs baseline: 1.5003x; 1.0323x over previous
import jax
import jax.numpy as jnp
from jax import lax
from jax.experimental import pallas as pl
from jax.experimental.pallas import tpu as pltpu
from jax.experimental.pallas import tpu_sc as plsc

N_USERS = 25000
N_ITEMS = 25000
N_NODES = N_USERS + N_ITEMS
D = 64
DQ = 16
NQ = D // DQ
NNZ = 800000

NS = 16
NC = 2
NROW = 50048
STRIPE = NROW // NS
E_GRP = 128
GRPS = 5
E_CHUNK = GRPS * E_GRP
CHUNKS = 80
EDGES_PER_TILE = CHUNKS * E_CHUNK
NNZ_PAD = EDGES_PER_TILE * NS
CROWS = NNZ_PAD // E_GRP
TROWS = CHUNKS * GRPS


_PRE_R = 2000


def _pre_body(t_ref, emb_ref, wa_ref, ba_ref, wp_ref, o0, o1, o2, o3):
    ph = t_ref[:] * wa_ref[:] + ba_ref[:]
    lane = lax.broadcasted_iota(jnp.int32, ph.shape, 1)
    val = jnp.where(lane == 0, ph, jnp.sin(ph))
    res = lax.dot_general(val, wp_ref[:], (((1,), (1,)), ((), ())),
                          preferred_element_type=jnp.float32)
    res = res + emb_ref[:]
    o0[:] = res[:, 0 * DQ:1 * DQ]
    o1[:] = res[:, 1 * DQ:2 * DQ]
    o2[:] = res[:, 2 * DQ:3 * DQ]
    o3[:] = res[:, 3 * DQ:4 * DQ]


def _preamble(t_all, emb_all, wa, ba, wproj):
    nb = N_NODES // _PRE_R
    qshape = jax.ShapeDtypeStruct((NROW, DQ), jnp.float32)
    qspec = pl.BlockSpec((_PRE_R, DQ), lambda i: (i, 0))
    return pl.pallas_call(
        _pre_body,
        grid=(nb,),
        in_specs=[
            pl.BlockSpec((_PRE_R, 1), lambda i: (i, 0)),
            pl.BlockSpec((_PRE_R, D), lambda i: (i, 0)),
            pl.BlockSpec((1, D), lambda i: (0, 0)),
            pl.BlockSpec((1, D), lambda i: (0, 0)),
            pl.BlockSpec((D, D), lambda i: (0, 0)),
        ],
        out_specs=[qspec] * NQ,
        out_shape=[qshape] * NQ,
    )(t_all, emb_all, wa, ba, wproj)


def _spmm_body(x0, cv4, rows2d, zstripe,
               o1, o2, o3,
               rowsv, ca, cb, ga, gb, acc, sem_i, sem_g, sem_s):
    c = lax.axis_index("c")
    s = lax.axis_index("s")
    row0 = s * STRIPE
    erow = s * TROWS

    pltpu.sync_copy(rows2d.at[pl.ds(erow, TROWS)], rowsv)

    def run_pass(src, dst, p):
        qrow = ((c * 2 + p) * NS + s) * (CHUNKS * 2 * GRPS)

        def idx_fetch(k, cbuf):
            pltpu.async_copy(cv4.at[pl.ds(qrow + k * 2 * GRPS, 2 * GRPS)],
                             cbuf, sem_i)

        def idx_wait(k, cbuf):
            pltpu.make_async_copy(cv4.at[pl.ds(qrow + k * 2 * GRPS,
                                               2 * GRPS)],
                                  cbuf, sem_i).wait()

        def g_issue(k, cbuf, gbuf):
            for j in range(GRPS):
                pltpu.async_copy(src.at[cbuf.at[j]], gbuf.at[j], sem_g)

        def g_wait(k, cbuf, gbuf):
            for j in range(GRPS):
                pltpu.make_async_copy(src.at[cbuf.at[j]], gbuf.at[j],
                                      sem_g).wait()

        def scale(k, cbuf, gbuf):
            for j in range(GRPS):
                def qb(q, _, j=j):
                    vv = plsc.bitcast(cbuf[GRPS + j, pl.ds(q * 16, 16)],
                                      jnp.float32)
                    for i in range(16):
                        e = q * 16 + i
                        gbuf[j, e, :] = gbuf[j, e, :] * vv[i]
                    return 0
                lax.fori_loop(0, E_GRP // 16, qb, 0)

        def s_issue(k, gbuf):
            for j in range(GRPS):
                pltpu.async_copy(gbuf.at[j], acc.at[rowsv.at[k * GRPS + j]],
                                 sem_s, add=True)

        def s_wait(k, gbuf):
            for j in range(GRPS):
                pltpu.make_async_copy(gbuf.at[j],
                                      acc.at[rowsv.at[k * GRPS + j]],
                                      sem_s).wait()

        def body(k, cur, nxt, first=False, last=False):
            cc, cg = cur
            nc, ng = nxt
            g_wait(k, cc, cg)
            if not last:
                idx_fetch(k + 1, nc)
            scale(k, cc, cg)
            if not first:
                s_wait(k - 1, ng)
            if not last:
                idx_wait(k + 1, nc)
                g_issue(k + 1, nc, ng)
            s_issue(k, cg)

        A = (ca, ga)
        B = (cb, gb)
        idx_fetch(0, ca)
        idx_wait(0, ca)
        g_issue(0, ca, ga)
        body(0, A, B, first=True)
        def steady(k2, carry):
            k = 2 * k2 + 1
            body(k, B, A)
            body(k + 1, A, B)
            return carry
        lax.fori_loop(0, (CHUNKS - 2) // 2, steady, 0)
        body(CHUNKS - 1, B, A, last=True)
        s_wait(CHUNKS - 1, gb)

    for src, dst in ((x0, o1), (o1, o2), (o2, o3)):
        def pbody(p, carry, src=src, dst=dst):
            pltpu.sync_copy(zstripe, acc.at[pl.ds(row0, STRIPE)])
            plsc.subcore_barrier()
            run_pass(src, dst, p)
            plsc.subcore_barrier()
            pltpu.sync_copy(
                acc.at[pl.ds(row0, STRIPE)],
                dst.at[pl.ds((c * 2 + p) * NROW + row0, STRIPE)])
            return carry
        lax.fori_loop(0, 2, pbody, 0)


def _spmm3(x0, cv4, rows2d, zstripe):
    mesh = plsc.VectorSubcoreMesh(core_axis_name="c", subcore_axis_name="s")
    xshape = jax.ShapeDtypeStruct((NQ * NROW, DQ), jnp.float32)
    f = pl.kernel(
        _spmm_body,
        out_type=(xshape, xshape, xshape),
        mesh=mesh,
        scratch_types=[
            pltpu.VMEM((TROWS, E_GRP), jnp.int32),
            pltpu.VMEM((2 * GRPS, E_GRP), jnp.int32),
            pltpu.VMEM((2 * GRPS, E_GRP), jnp.int32),
            pltpu.VMEM((GRPS, E_GRP, DQ), jnp.float32),
            pltpu.VMEM((GRPS, E_GRP, DQ), jnp.float32),
            pltpu.VMEM_SHARED((NROW, DQ), jnp.float32),
            pltpu.SemaphoreType.DMA,
            pltpu.SemaphoreType.DMA,
            pltpu.SemaphoreType.DMA,
        ],
        compiler_params=pltpu.CompilerParams(use_tc_tiling_on_sc=False,
                                             needs_layout_passes=False),
    )
    return f(x0, cv4, rows2d, zstripe)


_M_R = 1000


def _mean_body(a, b, c, d, ou, oi):
    i = pl.program_id(0)
    s = [(a[q] + b[q] + c[q] + d[q]) * 0.25 for q in range(NQ)]
    res = jnp.concatenate(s, axis=1)

    @pl.when(i < (N_USERS // _M_R))
    def _():
        ou[:] = res

    @pl.when(i >= (N_USERS // _M_R))
    def _():
        oi[:] = res


def _mean4(a, b, c, d):
    nu = N_USERS // _M_R
    ispec = pl.BlockSpec((NQ, _M_R, DQ), lambda i: (0, i, 0))
    uspec = pl.BlockSpec((_M_R, D), lambda i: (jnp.minimum(i, nu - 1), 0))
    ospec = pl.BlockSpec((_M_R, D), lambda i: (jnp.maximum(i - nu, 0), 0))
    oshape = jax.ShapeDtypeStruct((N_USERS, D), jnp.float32)
    return pl.pallas_call(
        _mean_body,
        grid=(N_NODES // _M_R,),
        in_specs=[ispec] * 4,
        out_specs=[uspec, ospec],
        out_shape=[oshape, oshape],
    )(a, b, c, d)


def kernel(user_recency, item_recency, adj_vals, user_emb, item_emb,
           w0, b0, w, b, Wproj, edge_index):
    t_all = jnp.concatenate([user_recency, item_recency]).reshape(N_NODES, 1)
    emb_all = jnp.concatenate([user_emb, item_emb], axis=0)
    wa = jnp.concatenate([w0, w]).reshape(1, D)
    ba = jnp.concatenate([b0, b]).reshape(1, D)

    xq = _preamble(t_all, emb_all, wa, ba, Wproj)
    x0 = jnp.concatenate(xq, axis=0)

    rows = edge_index[0].astype(jnp.int32)
    cols = edge_index[1].astype(jnp.int32)
    pad = NNZ_PAD - NNZ
    ipad = jnp.zeros((pad,), jnp.int32)
    rows2d = jnp.concatenate([rows, ipad]).reshape(CROWS, E_GRP)
    colsp = jnp.concatenate([cols, ipad])
    valsp = jnp.concatenate([adj_vals, jnp.zeros((pad,), jnp.float32)])
    cshift = (colsp[None, :]
              + (jnp.arange(NQ, dtype=jnp.int32) * NROW)[:, None])
    cshift = cshift.reshape(NQ, NS, CHUNKS, GRPS, E_GRP)
    vi = lax.bitcast_convert_type(valsp, jnp.int32)
    vi = jnp.broadcast_to(vi.reshape(1, NS, CHUNKS, GRPS, E_GRP),
                          (NQ, NS, CHUNKS, GRPS, E_GRP))
    cv4 = jnp.concatenate([cshift, vi], axis=3)
    cv4 = cv4.reshape(NQ * NS * CHUNKS * 2 * GRPS, E_GRP)
    zstripe = jnp.zeros((STRIPE, DQ), jnp.float32)

    x1, x2, x3 = _spmm3(x0, cv4, rows2d, zstripe)

    q3 = lambda v: v.reshape(NQ, NROW, DQ)
    u_out, i_out = _mean4(q3(x0), q3(x1), q3(x2), q3(x3))
    return u_out, i_out
